# baseline (speedup 1.0000x reference)
import jax
import jax.numpy as jnp
from jax.experimental import pallas as pl
from jax.experimental.pallas import tpu as pltpu

_BI = 400
_NBUF = 2


def _gcn_body(x_ref, adj_ref, w1_ref, b1_ref, w2_ref, b2_ref,
              out_ref, h_ref, buf_ref, s1_ref, s2_ref, sems):
    n = x_ref.shape[0]
    nb = n // _BI
    nfetch = 2 * nb - 1

    def fetch_block(f):
        b = jnp.where(f < nb, f, 2 * (nb - 1) - f)
        slot = jax.lax.rem(f, _NBUF)
        pltpu.make_async_copy(
            adj_ref.at[pl.ds(b * _BI, _BI), :],
            buf_ref.at[pl.ds(slot * _BI, _BI), :],
            sems.at[slot],
        ).start()

    for f in range(_NBUF):
        fetch_block(jnp.int32(f))

    s1_ref[...] = jnp.dot(x_ref[...], w1_ref[...],
                          preferred_element_type=jnp.float32)

    def step(t, _):
        c = jnp.where(t < nb, t, t - 1)
        slot = jax.lax.rem(c, _NBUF)
        b = jnp.where(t < nb, t, 2 * nb - 1 - t)
        rows = pl.ds(b * _BI, _BI)

        nxt = c + 1
        @pl.when(jnp.logical_and(jnp.logical_and(t != 0, t != nb),
                                 nxt < nfetch))
        def _():
            fetch_block(nxt)

        @pl.when(t != nb)
        def _():
            pltpu.make_async_copy(
                adj_ref.at[pl.ds(b * _BI, _BI), :],
                buf_ref.at[pl.ds(slot * _BI, _BI), :],
                sems.at[slot],
            ).wait()

        is_l1 = t < nb
        rhs = jnp.where(is_l1, s1_ref[...], s2_ref[...])
        acc = jnp.dot(buf_ref[pl.ds(slot * _BI, _BI), :], rhs,
                      preferred_element_type=jnp.float32)

        @pl.when(is_l1)
        def _():
            h = jnp.maximum(acc + b1_ref[...], 0.0)
            h_ref[rows, :] = h
            s2_ref[rows, :] = jnp.dot(h, w2_ref[...],
                                      preferred_element_type=jnp.float32)

        @pl.when(jnp.logical_not(is_l1))
        def _():
            logits = acc + b2_ref[...]
            m = jnp.max(logits, axis=-1, keepdims=True)
            lse = jnp.log(jnp.sum(jnp.exp(logits - m), axis=-1,
                                  keepdims=True)) + m
            out_ref[rows, :] = logits - lse

        return 0

    jax.lax.fori_loop(0, 2 * nb, step, 0)


def kernel(x, adj, W1, bias1, W2, bias2):
    n, nfeat = x.shape
    nhid = W1.shape[1]
    ncls = W2.shape[1]
    b1 = bias1.reshape(1, nhid)
    b2 = bias2.reshape(1, ncls)

    out, h = pl.pallas_call(
        _gcn_body,
        in_specs=[
            pl.BlockSpec(memory_space=pltpu.MemorySpace.VMEM),
            pl.BlockSpec(memory_space=pl.ANY),
            pl.BlockSpec(memory_space=pltpu.MemorySpace.VMEM),
            pl.BlockSpec(memory_space=pltpu.MemorySpace.VMEM),
            pl.BlockSpec(memory_space=pltpu.MemorySpace.VMEM),
            pl.BlockSpec(memory_space=pltpu.MemorySpace.VMEM),
        ],
        out_specs=[
            pl.BlockSpec(memory_space=pltpu.MemorySpace.VMEM),
            pl.BlockSpec(memory_space=pltpu.MemorySpace.VMEM),
        ],
        out_shape=[
            jax.ShapeDtypeStruct((n, ncls), jnp.float32),
            jax.ShapeDtypeStruct((n, nhid), jnp.float32),
        ],
        scratch_shapes=[
            pltpu.VMEM((_NBUF * _BI, n), jnp.float32),
            pltpu.VMEM((n, nhid), jnp.float32),
            pltpu.VMEM((n, ncls), jnp.float32),
            pltpu.SemaphoreType.DMA((_NBUF,)),
        ],
        compiler_params=pltpu.CompilerParams(
            vmem_limit_bytes=67108864,
        ),
    )(x, adj, W1, b1, W2, b2)

    return (out, h)


# speedup vs baseline: 1.0668x; 1.0668x over previous
import jax
import jax.numpy as jnp
from jax.experimental import pallas as pl
from jax.experimental.pallas import tpu as pltpu

_BI = 400
_N0 = 7
_N1 = 6
_W1 = 2560
_W2 = 4352


def _gcn_body(x_ref, adj_ref, w1_ref, b1_ref, w2_ref, b2_ref,
              out_ref, h_ref, buf_ref, s1_ref, s2_ref, sems):
    n = x_ref.shape[0]
    nb = n // _BI
    nfetch = 2 * nb - 1

    def fetch(k_block, col0, width, slot):
        pltpu.make_async_copy(
            adj_ref.at[pl.ds(k_block * _BI, _BI), pl.ds(col0, width)],
            buf_ref.at[pl.ds(slot * _BI, _BI), pl.ds(col0, width)],
            sems.at[slot],
        ).start()

    def wait(k_block, col0, width, slot):
        pltpu.make_async_copy(
            adj_ref.at[pl.ds(k_block * _BI, _BI), pl.ds(col0, width)],
            buf_ref.at[pl.ds(slot * _BI, _BI), pl.ds(col0, width)],
            sems.at[slot],
        ).wait()

    def fetch_col0(k):
        if k < nb:
            return 0, n
        b = 2 * (nb - 1) - k
        if b >= _N0 + _N1:
            return _W2, n - _W2
        if b >= _N0:
            return _W1, n - _W1
        return 0, n

    fetch(jnp.int32(0), 0, n, jnp.int32(0))
    fetch(jnp.int32(1), 0, n, jnp.int32(1))
    s1_ref[...] = jnp.dot(x_ref[...], w1_ref[...],
                          preferred_element_type=jnp.float32)

    def pass1_seg(lo, hi, partw):
        nxt_c0, nxt_w = fetch_col0(hi + 1)
        nxt_b = hi + 1 if hi + 1 < nb else 2 * (nb - 1) - (hi + 1)

        def body(t, _):
            slot = jax.lax.rem(t, 2)
            rows = pl.ds(t * _BI, _BI)

            @pl.when(jnp.logical_and(t > 0, t < hi))
            def _():
                fetch(t + 1, 0, n, jax.lax.rem(t + 1, 2))

            @pl.when(t == hi)
            def _():
                fetch(jnp.int32(nxt_b), nxt_c0, nxt_w,
                      jnp.int32((hi + 1) % 2))

            wait(t, 0, n, slot)
            blk = buf_ref[pl.ds(slot * _BI, _BI), :]
            acc = jnp.dot(blk, s1_ref[...],
                          preferred_element_type=jnp.float32)
            h = jnp.maximum(acc + b1_ref[...], 0.0)
            h_ref[rows, :] = h
            s2_ref[rows, :] = jnp.dot(h, w2_ref[...],
                                      preferred_element_type=jnp.float32)
            if partw:
                out_ref[rows, :] = jnp.dot(
                    buf_ref[pl.ds(slot * _BI, _BI), pl.ds(0, partw)],
                    s2_ref[pl.ds(0, partw), :],
                    preferred_element_type=jnp.float32)
            return 0

        jax.lax.fori_loop(lo, hi + 1, body, 0)

    def pass2_compute(b, slot, col0, has_partial):
        rows = pl.ds(b * _BI, _BI)
        tail = jnp.dot(
            buf_ref[pl.ds(slot * _BI, _BI), pl.ds(col0, n - col0)],
            s2_ref[pl.ds(col0, n - col0), :],
            preferred_element_type=jnp.float32)
        logits = tail + b2_ref[...]
        if has_partial:
            logits = logits + out_ref[rows, :]
        m = jnp.max(logits, axis=-1, keepdims=True)
        lse = jnp.log(jnp.sum(jnp.exp(logits - m), axis=-1,
                              keepdims=True)) + m
        out_ref[rows, :] = logits - lse

    def pass2_seg(lo, hi, col0, has_partial):
        if hi < nfetch:
            nxt_c0, nxt_w = fetch_col0(hi)
            nxt_b = 2 * (nb - 1) - hi
        else:
            nxt_c0 = nxt_w = nxt_b = None

        def body(t, _):
            c = t - 1
            slot = jax.lax.rem(c, 2)
            b = 2 * nb - 1 - t

            @pl.when(t < hi)
            def _():
                fetch(47 - c, col0, n - col0, jax.lax.rem(c + 1, 2))

            if nxt_b is not None:
                @pl.when(t == hi)
                def _():
                    fetch(jnp.int32(nxt_b), nxt_c0, nxt_w,
                          jnp.int32(hi % 2))

            wait(b, col0, n - col0, slot)
            pass2_compute(b, slot, col0, has_partial)
            return 0

        jax.lax.fori_loop(lo, hi + 1, body, 0)

    pass1_seg(0, _N0 - 1, 0)
    pass1_seg(_N0, _N0 + _N1 - 1, _W1)
    pass1_seg(_N0 + _N1, nb - 1, _W2)

    pass2_compute(jnp.int32(nb - 1), jnp.int32(0), _W2, True)
    pass2_seg(26, 36, _W2, True)
    pass2_seg(37, 42, _W1, True)
    pass2_seg(43, 49, 0, False)


def kernel(x, adj, W1, bias1, W2, bias2):
    n, nfeat = x.shape
    nhid = W1.shape[1]
    ncls = W2.shape[1]
    b1 = bias1.reshape(1, nhid)
    b2 = bias2.reshape(1, ncls)

    out, h = pl.pallas_call(
        _gcn_body,
        in_specs=[
            pl.BlockSpec(memory_space=pltpu.MemorySpace.VMEM),
            pl.BlockSpec(memory_space=pl.ANY),
            pl.BlockSpec(memory_space=pltpu.MemorySpace.VMEM),
            pl.BlockSpec(memory_space=pltpu.MemorySpace.VMEM),
            pl.BlockSpec(memory_space=pltpu.MemorySpace.VMEM),
            pl.BlockSpec(memory_space=pltpu.MemorySpace.VMEM),
        ],
        out_specs=[
            pl.BlockSpec(memory_space=pltpu.MemorySpace.VMEM),
            pl.BlockSpec(memory_space=pltpu.MemorySpace.VMEM),
        ],
        out_shape=[
            jax.ShapeDtypeStruct((n, ncls), jnp.float32),
            jax.ShapeDtypeStruct((n, nhid), jnp.float32),
        ],
        scratch_shapes=[
            pltpu.VMEM((2 * _BI, n), jnp.float32),
            pltpu.VMEM((n, nhid), jnp.float32),
            pltpu.VMEM((n, ncls), jnp.float32),
            pltpu.SemaphoreType.DMA((2,)),
        ],
        compiler_params=pltpu.CompilerParams(
            vmem_limit_bytes=67108864,
        ),
    )(x, adj, W1, b1, W2, b2)

    return (out, h)


# speedup vs baseline: 1.0701x; 1.0031x over previous
import jax
import jax.numpy as jnp
from jax.experimental import pallas as pl
from jax.experimental.pallas import tpu as pltpu

_BI = 400
_N0 = 7
_N1 = 6
_W1 = 2688
_W2 = 4992


def _gcn_body(x_ref, adj_ref, w1_ref, b1_ref, w2_ref, b2_ref,
              out_ref, h_ref, buf_ref, s1_ref, s2_ref, sems):
    n = x_ref.shape[0]
    nb = n // _BI
    nfetch = 2 * nb - 1

    def fetch(k_block, col0, width, slot):
        pltpu.make_async_copy(
            adj_ref.at[pl.ds(k_block * _BI, _BI), pl.ds(col0, width)],
            buf_ref.at[pl.ds(slot * _BI, _BI), pl.ds(col0, width)],
            sems.at[slot],
        ).start()

    def wait(k_block, col0, width, slot):
        pltpu.make_async_copy(
            adj_ref.at[pl.ds(k_block * _BI, _BI), pl.ds(col0, width)],
            buf_ref.at[pl.ds(slot * _BI, _BI), pl.ds(col0, width)],
            sems.at[slot],
        ).wait()

    def fetch_col0(k):
        if k < nb:
            return 0, n
        b = 2 * (nb - 1) - k
        if b >= _N0 + _N1:
            return _W2, n - _W2
        if b >= _N0:
            return _W1, n - _W1
        return 0, n

    fetch(jnp.int32(0), 0, n, jnp.int32(0))
    fetch(jnp.int32(1), 0, n, jnp.int32(1))
    s1_ref[...] = jnp.dot(x_ref[...], w1_ref[...],
                          preferred_element_type=jnp.float32)

    def pass1_seg(lo, hi, partw):
        nxt_c0, nxt_w = fetch_col0(hi + 1)
        nxt_b = hi + 1 if hi + 1 < nb else 2 * (nb - 1) - (hi + 1)

        def body(t, _):
            slot = jax.lax.rem(t, 2)
            rows = pl.ds(t * _BI, _BI)

            @pl.when(jnp.logical_and(t > 0, t < hi))
            def _():
                fetch(t + 1, 0, n, jax.lax.rem(t + 1, 2))

            @pl.when(t == hi)
            def _():
                fetch(jnp.int32(nxt_b), nxt_c0, nxt_w,
                      jnp.int32((hi + 1) % 2))

            wait(t, 0, n, slot)
            blk = buf_ref[pl.ds(slot * _BI, _BI), :]
            acc = jnp.dot(blk, s1_ref[...],
                          preferred_element_type=jnp.float32)
            h = jnp.maximum(acc + b1_ref[...], 0.0)
            h_ref[rows, :] = h
            s2_ref[rows, :] = jnp.dot(h, w2_ref[...],
                                      preferred_element_type=jnp.float32)
            if partw:
                out_ref[rows, :] = jnp.dot(
                    buf_ref[pl.ds(slot * _BI, _BI), pl.ds(0, partw)],
                    s2_ref[pl.ds(0, partw), :],
                    preferred_element_type=jnp.float32)
            return 0

        jax.lax.fori_loop(lo, hi + 1, body, 0)

    def pass2_compute(b, slot, col0, has_partial):
        rows = pl.ds(b * _BI, _BI)
        tail = jnp.dot(
            buf_ref[pl.ds(slot * _BI, _BI), pl.ds(col0, n - col0)],
            s2_ref[pl.ds(col0, n - col0), :],
            preferred_element_type=jnp.float32)
        logits = tail + b2_ref[...]
        if has_partial:
            logits = logits + out_ref[rows, :]
        m = jnp.max(logits, axis=-1, keepdims=True)
        lse = jnp.log(jnp.sum(jnp.exp(logits - m), axis=-1,
                              keepdims=True)) + m
        out_ref[rows, :] = logits - lse

    def pass2_seg(lo, hi, col0, has_partial):
        if hi < nfetch:
            nxt_c0, nxt_w = fetch_col0(hi)
            nxt_b = 2 * (nb - 1) - hi
        else:
            nxt_c0 = nxt_w = nxt_b = None

        def body(t, _):
            c = t - 1
            slot = jax.lax.rem(c, 2)
            b = 2 * nb - 1 - t

            @pl.when(t < hi)
            def _():
                fetch(47 - c, col0, n - col0, jax.lax.rem(c + 1, 2))

            if nxt_b is not None:
                @pl.when(t == hi)
                def _():
                    fetch(jnp.int32(nxt_b), nxt_c0, nxt_w,
                          jnp.int32(hi % 2))

            wait(b, col0, n - col0, slot)
            pass2_compute(b, slot, col0, has_partial)
            return 0

        jax.lax.fori_loop(lo, hi + 1, body, 0)

    pass1_seg(0, _N0 - 1, 0)
    pass1_seg(_N0, _N0 + _N1 - 1, _W1)
    pass1_seg(_N0 + _N1, nb - 1, _W2)

    pass2_compute(jnp.int32(nb - 1), jnp.int32(0), _W2, True)
    pass2_seg(26, 36, _W2, True)
    pass2_seg(37, 42, _W1, True)
    pass2_seg(43, 49, 0, False)


def kernel(x, adj, W1, bias1, W2, bias2):
    n, nfeat = x.shape
    nhid = W1.shape[1]
    ncls = W2.shape[1]
    b1 = bias1.reshape(1, nhid)
    b2 = bias2.reshape(1, ncls)

    out, h = pl.pallas_call(
        _gcn_body,
        in_specs=[
            pl.BlockSpec(memory_space=pltpu.MemorySpace.VMEM),
            pl.BlockSpec(memory_space=pl.ANY),
            pl.BlockSpec(memory_space=pltpu.MemorySpace.VMEM),
            pl.BlockSpec(memory_space=pltpu.MemorySpace.VMEM),
            pl.BlockSpec(memory_space=pltpu.MemorySpace.VMEM),
            pl.BlockSpec(memory_space=pltpu.MemorySpace.VMEM),
        ],
        out_specs=[
            pl.BlockSpec(memory_space=pltpu.MemorySpace.VMEM),
            pl.BlockSpec(memory_space=pltpu.MemorySpace.VMEM),
        ],
        out_shape=[
            jax.ShapeDtypeStruct((n, ncls), jnp.float32),
            jax.ShapeDtypeStruct((n, nhid), jnp.float32),
        ],
        scratch_shapes=[
            pltpu.VMEM((2 * _BI, n), jnp.float32),
            pltpu.VMEM((n, nhid), jnp.float32),
            pltpu.VMEM((n, ncls), jnp.float32),
            pltpu.SemaphoreType.DMA((2,)),
        ],
        compiler_params=pltpu.CompilerParams(
            vmem_limit_bytes=67108864,
        ),
    )(x, adj, W1, b1, W2, b2)

    return (out, h)


# speedup vs baseline: 1.0837x; 1.0128x over previous
import jax
import jax.numpy as jnp
from jax.experimental import pallas as pl
from jax.experimental.pallas import tpu as pltpu

_BI = 400
_GROUPS = ((4, 0), (5, 1536), (5, 3584), (11, 4992))


def _gcn_body(x_ref, adj_ref, w1_ref, b1_ref, w2_ref, b2_ref,
              out_ref, h_ref, buf_ref, s1_ref, s2_ref, sems):
    n = x_ref.shape[0]
    nb = n // _BI
    nfetch = 2 * nb - 1

    def fetch(k_block, col0, width, slot):
        pltpu.make_async_copy(
            adj_ref.at[pl.ds(k_block * _BI, _BI), pl.ds(col0, width)],
            buf_ref.at[pl.ds(slot * _BI, _BI), pl.ds(col0, width)],
            sems.at[slot],
        ).start()

    def wait(k_block, col0, width, slot):
        pltpu.make_async_copy(
            adj_ref.at[pl.ds(k_block * _BI, _BI), pl.ds(col0, width)],
            buf_ref.at[pl.ds(slot * _BI, _BI), pl.ds(col0, width)],
            sems.at[slot],
        ).wait()

    def group_w(b):
        lo = 0
        for cnt, w in _GROUPS:
            if b < lo + cnt:
                return w
            lo += cnt
        raise AssertionError

    def fetch_col0(k):
        if k < nb:
            return 0, n
        w = group_w(2 * (nb - 1) - k)
        return w, n - w

    fetch(jnp.int32(0), 0, n, jnp.int32(0))
    fetch(jnp.int32(1), 0, n, jnp.int32(1))
    s1_ref[...] = jnp.dot(x_ref[...], w1_ref[...],
                          preferred_element_type=jnp.float32)

    def pass1_seg(lo, hi, partw):
        nxt_c0, nxt_w = fetch_col0(hi + 1)
        nxt_b = hi + 1 if hi + 1 < nb else 2 * (nb - 1) - (hi + 1)

        def body(t, _):
            slot = jax.lax.rem(t, 2)
            rows = pl.ds(t * _BI, _BI)

            @pl.when(jnp.logical_and(t > 0, t < hi))
            def _():
                fetch(t + 1, 0, n, jax.lax.rem(t + 1, 2))

            @pl.when(t == hi)
            def _():
                fetch(jnp.int32(nxt_b), nxt_c0, nxt_w,
                      jnp.int32((hi + 1) % 2))

            wait(t, 0, n, slot)
            blk = buf_ref[pl.ds(slot * _BI, _BI), :]
            acc = jnp.dot(blk, s1_ref[...],
                          preferred_element_type=jnp.float32)
            h = jnp.maximum(acc + b1_ref[...], 0.0)
            h_ref[rows, :] = h
            s2_ref[rows, :] = jnp.dot(h, w2_ref[...],
                                      preferred_element_type=jnp.float32)
            if partw:
                out_ref[rows, :] = jnp.dot(
                    buf_ref[pl.ds(slot * _BI, _BI), pl.ds(0, partw)],
                    s2_ref[pl.ds(0, partw), :],
                    preferred_element_type=jnp.float32)
            return 0

        jax.lax.fori_loop(lo, hi + 1, body, 0)

    def pass2_compute(b, slot, col0, has_partial):
        rows = pl.ds(b * _BI, _BI)
        tail = jnp.dot(
            buf_ref[pl.ds(slot * _BI, _BI), pl.ds(col0, n - col0)],
            s2_ref[pl.ds(col0, n - col0), :],
            preferred_element_type=jnp.float32)
        logits = tail + b2_ref[...]
        if has_partial:
            logits = logits + out_ref[rows, :]
        m = jnp.max(logits, axis=-1, keepdims=True)
        lse = jnp.log(jnp.sum(jnp.exp(logits - m), axis=-1,
                              keepdims=True)) + m
        out_ref[rows, :] = logits - lse

    def pass2_seg(lo, hi, col0, has_partial):
        if hi < nfetch:
            nxt_c0, nxt_w = fetch_col0(hi)
            nxt_b = 2 * (nb - 1) - hi
        else:
            nxt_c0 = nxt_w = nxt_b = None

        def body(t, _):
            c = t - 1
            slot = jax.lax.rem(c, 2)
            b = 2 * nb - 1 - t

            @pl.when(t < hi)
            def _():
                fetch(47 - c, col0, n - col0, jax.lax.rem(c + 1, 2))

            if nxt_b is not None:
                @pl.when(t == hi)
                def _():
                    fetch(jnp.int32(nxt_b), nxt_c0, nxt_w,
                          jnp.int32(hi % 2))

            wait(b, col0, n - col0, slot)
            pass2_compute(b, slot, col0, has_partial)
            return 0

        jax.lax.fori_loop(lo, hi + 1, body, 0)

    lo = 0
    for cnt, w in _GROUPS:
        pass1_seg(lo, lo + cnt - 1, w)
        lo += cnt

    wlast = _GROUPS[-1][1]
    pass2_compute(jnp.int32(nb - 1), jnp.int32(0), wlast, wlast > 0)
    t0 = 26
    first = True
    for cnt, w in reversed(_GROUPS):
        if first:
            cnt -= 1
            first = False
        pass2_seg(t0, t0 + cnt - 1, w, w > 0)
        t0 += cnt


def kernel(x, adj, W1, bias1, W2, bias2):
    n, nfeat = x.shape
    nhid = W1.shape[1]
    ncls = W2.shape[1]
    b1 = bias1.reshape(1, nhid)
    b2 = bias2.reshape(1, ncls)

    out, h = pl.pallas_call(
        _gcn_body,
        in_specs=[
            pl.BlockSpec(memory_space=pltpu.MemorySpace.VMEM),
            pl.BlockSpec(memory_space=pl.ANY),
            pl.BlockSpec(memory_space=pltpu.MemorySpace.VMEM),
            pl.BlockSpec(memory_space=pltpu.MemorySpace.VMEM),
            pl.BlockSpec(memory_space=pltpu.MemorySpace.VMEM),
            pl.BlockSpec(memory_space=pltpu.MemorySpace.VMEM),
        ],
        out_specs=[
            pl.BlockSpec(memory_space=pltpu.MemorySpace.VMEM),
            pl.BlockSpec(memory_space=pltpu.MemorySpace.VMEM),
        ],
        out_shape=[
            jax.ShapeDtypeStruct((n, ncls), jnp.float32),
            jax.ShapeDtypeStruct((n, nhid), jnp.float32),
        ],
        scratch_shapes=[
            pltpu.VMEM((2 * _BI, n), jnp.float32),
            pltpu.VMEM((n, nhid), jnp.float32),
            pltpu.VMEM((n, ncls), jnp.float32),
            pltpu.SemaphoreType.DMA((2,)),
        ],
        compiler_params=pltpu.CompilerParams(
            vmem_limit_bytes=67108864,
        ),
    )(x, adj, W1, b1, W2, b2)

    return (out, h)
